# baseline (speedup 1.0000x reference)
import jax
import jax.numpy as jnp
from jax import lax
from jax.experimental import pallas as pl
from jax.experimental.pallas import tpu as pltpu
from jax.experimental.pallas import tpu_sc as plsc

NC = 2
NS = 16
NW = NC * NS
K = 128


def _mesh():
    return plsc.VectorSubcoreMesh(
        core_axis_name="c", subcore_axis_name="s", num_cores=NC, num_subcores=NS
    )


def _deg_body(ch, acc_n, n):
    stripe = acc_n // NS

    def body(rows_hbm, cols_hbm, cnt_hbm, hl_hbm, ridx, cidx, vones, vhl, acc_c, acc_h):
        c = lax.axis_index("c")
        s = lax.axis_index("s")
        wid = s * NC + c
        zeros16 = jnp.zeros((16,), jnp.float32)
        ones16 = jnp.ones((16,), jnp.float32)

        def zv(i, carry):
            vhl[pl.ds(i * 16, 16)] = zeros16
            return carry

        lax.fori_loop(0, K // 16, zv, 0)
        for t in range(stripe // K):
            pltpu.sync_copy(vhl, acc_c.at[pl.ds(s * stripe + t * K, K)])
            pltpu.sync_copy(vhl, acc_h.at[pl.ds(s * stripe + t * K, K)])

        def sv(i, carry):
            vones[pl.ds(i * 16, 16)] = ones16
            return carry

        lax.fori_loop(0, K // 16, sv, 0)
        plsc.subcore_barrier()

        pltpu.sync_copy(rows_hbm.at[wid], ridx)
        pltpu.sync_copy(cols_hbm.at[wid], cidx)

        def chunk(j, carry):
            def inner(i, icarry):
                r16 = ridx[j, pl.ds(i * 16, 16)]
                c16 = cidx[j, pl.ds(i * 16, 16)]
                vhl[pl.ds(i * 16, 16)] = jnp.where(r16 == c16, ones16, zeros16)
                return icarry

            lax.fori_loop(0, K // 16, inner, 0)
            pltpu.sync_copy(vones, acc_c.at[cidx.at[j]], add=True)
            pltpu.sync_copy(vhl, acc_h.at[cidx.at[j]], add=True)
            return carry

        lax.fori_loop(0, ch, chunk, 0)
        plsc.subcore_barrier()

        pltpu.sync_copy(
            acc_c.at[pl.ds(s * stripe, stripe)],
            cnt_hbm.at[c, pl.ds(s * stripe, stripe)],
        )
        pltpu.sync_copy(
            acc_h.at[pl.ds(s * stripe, stripe)],
            hl_hbm.at[c, pl.ds(s * stripe, stripe)],
        )

    return body


def _agg_body(ch, acc_n, n, h):
    stripe = acc_n // NS

    assert ch >= 8

    def body(
        yp_hbm, rows_hbm, cols_hbm, out_hbm, ridx, cidx, gbuf, acc,
        g0, g1, g2, g3, s0, s1, s2, s3,
    ):
        c = lax.axis_index("c")
        s = lax.axis_index("s")
        wid = s * NC + c
        zeros16 = jnp.zeros((16,), jnp.float32)
        gsems = (g0, g1, g2, g3)
        ssems = (s0, s1, s2, s3)

        def zg(i, carry):
            gbuf[0, i // (h // 16), pl.ds((i % (h // 16)) * 16, 16)] = zeros16
            return carry

        lax.fori_loop(0, K * h // 16, zg, 0)

        for t in range(stripe // K):
            pltpu.sync_copy(gbuf.at[0], acc.at[pl.ds(s * stripe + t * K, K)])
        plsc.subcore_barrier()

        pltpu.sync_copy(rows_hbm.at[wid], ridx)
        pltpu.sync_copy(cols_hbm.at[wid], cidx)

        def issue_g(j, b):
            pltpu.async_copy(yp_hbm.at[ridx.at[j]], gbuf.at[b], gsems[b])

        def wait_g(j, b):
            pltpu.make_async_copy(yp_hbm.at[ridx.at[j]], gbuf.at[b], gsems[b]).wait()

        def issue_s(j, b):
            pltpu.async_copy(gbuf.at[b], acc.at[cidx.at[j]], ssems[b], add=True)

        def wait_s(j, b):
            pltpu.make_async_copy(gbuf.at[b], acc.at[cidx.at[j]], ssems[b]).wait()

        issue_g(0, 0)
        issue_g(1, 1)
        issue_g(2, 2)
        wait_g(0, 0)
        issue_s(0, 0)
        issue_g(3, 3)
        wait_g(1, 1)
        issue_s(1, 1)

        steady = range(2, ch - 2)
        nquad = len(steady) // 4

        def quad(q, carry):
            j0 = 2 + 4 * q
            for r in range(4):
                j = j0 + r
                b = (2 + r) % 4
                wait_s(j - 2, r % 4)
                issue_g(j + 2, r % 4)
                wait_g(j, b)
                issue_s(j, b)
            return carry

        lax.fori_loop(0, nquad, quad, 0)
        for j in range(2 + 4 * nquad, ch - 2):
            wait_s(j - 2, (j - 2) % 4)
            issue_g(j + 2, (j + 2) % 4)
            wait_g(j, j % 4)
            issue_s(j, j % 4)
        for j in range(ch - 2, ch):
            wait_g(j, j % 4)
            issue_s(j, j % 4)
        for j in range(ch - 4, ch):
            wait_s(j, j % 4)

        plsc.subcore_barrier()
        pltpu.sync_copy(
            acc.at[pl.ds(s * stripe, stripe)],
            out_hbm.at[c, pl.ds(s * stripe, stripe)],
        )

    return body


def _norm_factors(cnt_ref, hl_ref):
    cnt = cnt_ref[0, :] + cnt_ref[1, :]
    hl = hl_ref[0, :] + hl_ref[1, :]
    loopw = jnp.where(hl == 0.0, 1.0, 0.0)
    dis = lax.rsqrt(cnt + loopw)
    bn = cnt.shape[0]
    return jnp.reshape(dis, (bn, 1)), jnp.reshape(loopw, (bn, 1))


def _mm_scale_body(x_ref, w_ref, cnt_ref, hl_ref, yp_ref, st_ref):
    dis, loopw = _norm_factors(cnt_ref, hl_ref)
    y = jnp.dot(x_ref[:, :], w_ref[:, :], preferred_element_type=jnp.float32)
    yp_ref[:, :] = y * dis
    st_ref[:, :] = y * (loopw * dis * dis)


def _layer_body(agg_ref, st_ref, cnt_ref, hl_ref, b_ref, w_ref, yp_ref, st2_ref):
    dis, loopw = _norm_factors(cnt_ref, hl_ref)
    agg = agg_ref[0, :, :] + agg_ref[1, :, :]
    hcur = jnp.maximum(agg * dis + st_ref[:, :] + b_ref[:, :], 0.0)
    y2 = jnp.dot(hcur, w_ref[:, :], preferred_element_type=jnp.float32)
    yp_ref[:, :] = y2 * dis
    st2_ref[:, :] = y2 * (loopw * dis * dis)


def _head_body(agg_ref, st_ref, cnt_ref, hl_ref, b_ref, wl_ref, bl_ref, o_ref):
    dis, loopw = _norm_factors(cnt_ref, hl_ref)
    agg = agg_ref[0, :, :] + agg_ref[1, :, :]
    hcur = jnp.maximum(agg * dis + st_ref[:, :] + b_ref[:, :], 0.0)
    o = jnp.dot(hcur, wl_ref[:, :], preferred_element_type=jnp.float32) + bl_ref[:, :]
    m = jnp.max(o, axis=1, keepdims=True)
    lse = m + jnp.log(jnp.sum(jnp.exp(o - m), axis=1, keepdims=True))
    o_ref[:, :] = o - lse


def kernel(x, edge_index, W1, b1, W2, b2, Wl, bl):
    n, f_in = x.shape
    e = edge_index.shape[1]
    h = W1.shape[1]
    c_out = Wl.shape[1]
    assert n % NS == 0 and h % 16 == 0

    ch = -(-e // (NW * K))
    e_pad = NW * ch * K
    acc_n = -(-(n + 1) // (NS * K)) * (NS * K)

    ei = edge_index.astype(jnp.int32)
    rows = jnp.concatenate([ei[0], jnp.zeros((e_pad - e,), jnp.int32)])
    cols = jnp.concatenate([ei[1], jnp.full((e_pad - e,), n, jnp.int32)])
    rows_l = rows.reshape(NW, ch, K)
    cols_l = cols.reshape(NW, ch, K)

    mesh = _mesh()

    cnt0, hl0 = pl.kernel(
        _deg_body(ch, acc_n, n),
        out_type=[
            jax.ShapeDtypeStruct((NC, acc_n), jnp.float32),
            jax.ShapeDtypeStruct((NC, acc_n), jnp.float32),
        ],
        mesh=mesh,
        scratch_types=[
            pltpu.VMEM((ch, K), jnp.int32),
            pltpu.VMEM((ch, K), jnp.int32),
            pltpu.VMEM((K,), jnp.float32),
            pltpu.VMEM((K,), jnp.float32),
            pltpu.VMEM_SHARED((acc_n,), jnp.float32),
            pltpu.VMEM_SHARED((acc_n,), jnp.float32),
        ],
    )(rows_l, cols_l)

    bn = 1280
    assert acc_n % bn == 0
    grid = (acc_n // bn,)
    y1p, st1 = pl.pallas_call(
        _mm_scale_body,
        grid=grid,
        in_specs=[
            pl.BlockSpec((bn, f_in), lambda i: (i, 0)),
            pl.BlockSpec((f_in, h), lambda i: (0, 0)),
            pl.BlockSpec((NC, bn), lambda i: (0, i)),
            pl.BlockSpec((NC, bn), lambda i: (0, i)),
        ],
        out_specs=[
            pl.BlockSpec((bn, h), lambda i: (i, 0)),
            pl.BlockSpec((bn, h), lambda i: (i, 0)),
        ],
        out_shape=[
            jax.ShapeDtypeStruct((acc_n, h), jnp.float32),
            jax.ShapeDtypeStruct((acc_n, h), jnp.float32),
        ],
    )(x, W1, cnt0, hl0)

    agg_call = pl.kernel(
        _agg_body(ch, acc_n, n, h),
        out_type=jax.ShapeDtypeStruct((NC, acc_n, h), jnp.float32),
        mesh=mesh,
        compiler_params=pltpu.CompilerParams(use_tc_tiling_on_sc=False),
        scratch_types=[
            pltpu.VMEM((ch, K), jnp.int32),
            pltpu.VMEM((ch, K), jnp.int32),
            pltpu.VMEM((4, K, h), jnp.float32),
            pltpu.VMEM_SHARED((acc_n, h), jnp.float32),
        ]
        + [pltpu.SemaphoreType.DMA] * 8,
    )

    agg1 = agg_call(y1p, rows_l, cols_l)

    y2p, st2 = pl.pallas_call(
        _layer_body,
        grid=grid,
        in_specs=[
            pl.BlockSpec((NC, bn, h), lambda i: (0, i, 0)),
            pl.BlockSpec((bn, h), lambda i: (i, 0)),
            pl.BlockSpec((NC, bn), lambda i: (0, i)),
            pl.BlockSpec((NC, bn), lambda i: (0, i)),
            pl.BlockSpec((1, h), lambda i: (0, 0)),
            pl.BlockSpec((h, h), lambda i: (0, 0)),
        ],
        out_specs=[
            pl.BlockSpec((bn, h), lambda i: (i, 0)),
            pl.BlockSpec((bn, h), lambda i: (i, 0)),
        ],
        out_shape=[
            jax.ShapeDtypeStruct((acc_n, h), jnp.float32),
            jax.ShapeDtypeStruct((acc_n, h), jnp.float32),
        ],
    )(agg1, st1, cnt0, hl0, b1.reshape(1, h), W2)

    agg2 = agg_call(y2p, rows_l, cols_l)

    out = pl.pallas_call(
        _head_body,
        grid=grid,
        in_specs=[
            pl.BlockSpec((NC, bn, h), lambda i: (0, i, 0)),
            pl.BlockSpec((bn, h), lambda i: (i, 0)),
            pl.BlockSpec((NC, bn), lambda i: (0, i)),
            pl.BlockSpec((NC, bn), lambda i: (0, i)),
            pl.BlockSpec((1, h), lambda i: (0, 0)),
            pl.BlockSpec((h, c_out), lambda i: (0, 0)),
            pl.BlockSpec((1, c_out), lambda i: (0, 0)),
        ],
        out_specs=pl.BlockSpec((bn, c_out), lambda i: (i, 0)),
        out_shape=jax.ShapeDtypeStruct((n, c_out), jnp.float32),
    )(agg2, st2, cnt0, hl0, b2.reshape(1, h), Wl, bl.reshape(1, c_out))

    return out


# speedup vs baseline: 1.4463x; 1.4463x over previous
import jax
import jax.numpy as jnp
from jax import lax
from jax.experimental import pallas as pl
from jax.experimental.pallas import tpu as pltpu
from jax.experimental.pallas import tpu_sc as plsc

NC = 2
NS = 16
NW = NC * NS
K = 128


def _mesh():
    return plsc.VectorSubcoreMesh(
        core_axis_name="c", subcore_axis_name="s", num_cores=NC, num_subcores=NS
    )


def _deg_body(ch0, ch1, acc_n, n):
    stripe = acc_n // NS

    def body(rows_hbm, cols_hbm, cnt_hbm, hl_hbm, ridx, cidx, vones, vhl, acc_c, acc_h):
        c = lax.axis_index("c")
        s = lax.axis_index("s")
        wid = s * NC + c
        ch_c = jnp.where(c == 0, ch0, ch1)
        zeros16 = jnp.zeros((16,), jnp.float32)
        ones16 = jnp.ones((16,), jnp.float32)

        def zv(i, carry):
            vhl[pl.ds(i * 16, 16)] = zeros16
            return carry

        lax.fori_loop(0, K // 16, zv, 0)
        for t in range(stripe // K):
            pltpu.sync_copy(vhl, acc_c.at[pl.ds(s * stripe + t * K, K)])
            pltpu.sync_copy(vhl, acc_h.at[pl.ds(s * stripe + t * K, K)])

        def sv(i, carry):
            vones[pl.ds(i * 16, 16)] = ones16
            return carry

        lax.fori_loop(0, K // 16, sv, 0)
        plsc.subcore_barrier()

        pltpu.sync_copy(rows_hbm.at[wid], ridx)
        pltpu.sync_copy(cols_hbm.at[wid], cidx)

        def chunk(j, carry):
            def inner(i, icarry):
                r16 = ridx[j, pl.ds(i * 16, 16)]
                c16 = cidx[j, pl.ds(i * 16, 16)]
                vhl[pl.ds(i * 16, 16)] = jnp.where(r16 == c16, ones16, zeros16)
                return icarry

            lax.fori_loop(0, K // 16, inner, 0)
            pltpu.sync_copy(vones, acc_c.at[cidx.at[j]], add=True)
            pltpu.sync_copy(vhl, acc_h.at[cidx.at[j]], add=True)
            return carry

        lax.fori_loop(0, ch_c, chunk, 0)
        plsc.subcore_barrier()

        pltpu.sync_copy(
            acc_c.at[pl.ds(s * stripe, stripe)],
            cnt_hbm.at[c, pl.ds(s * stripe, stripe)],
        )
        pltpu.sync_copy(
            acc_h.at[pl.ds(s * stripe, stripe)],
            hl_hbm.at[c, pl.ds(s * stripe, stripe)],
        )

    return body


def _agg_body(ch0, ch1, acc_n, n, h):
    stripe = acc_n // NS

    assert ch0 >= 8 and ch1 >= 8

    def body(
        yp_hbm, rows_hbm, cols_hbm, out_hbm, ridx, cidx, gbuf, acc,
        g0, g1, g2, g3, s0, s1, s2, s3,
    ):
        c = lax.axis_index("c")
        s = lax.axis_index("s")
        wid = s * NC + c
        zeros16 = jnp.zeros((16,), jnp.float32)
        gsems = (g0, g1, g2, g3)
        ssems = (s0, s1, s2, s3)

        def zg(i, carry):
            gbuf[0, i // (h // 16), pl.ds((i % (h // 16)) * 16, 16)] = zeros16
            return carry

        lax.fori_loop(0, K * h // 16, zg, 0)

        for t in range(stripe // K):
            pltpu.sync_copy(gbuf.at[0], acc.at[pl.ds(s * stripe + t * K, K)])
        plsc.subcore_barrier()

        pltpu.sync_copy(rows_hbm.at[wid], ridx)
        pltpu.sync_copy(cols_hbm.at[wid], cidx)

        def issue_g(j, b):
            pltpu.async_copy(yp_hbm.at[ridx.at[j]], gbuf.at[b], gsems[b])

        def wait_g(j, b):
            pltpu.make_async_copy(yp_hbm.at[ridx.at[j]], gbuf.at[b], gsems[b]).wait()

        def issue_s(j, b):
            pltpu.async_copy(gbuf.at[b], acc.at[cidx.at[j]], ssems[b], add=True)

        def wait_s(j, b):
            pltpu.make_async_copy(gbuf.at[b], acc.at[cidx.at[j]], ssems[b]).wait()

        def schedule(ch):
            issue_g(0, 0)
            issue_g(1, 1)
            issue_g(2, 2)
            wait_g(0, 0)
            issue_s(0, 0)
            issue_g(3, 3)
            wait_g(1, 1)
            issue_s(1, 1)

            steady = range(2, ch - 2)
            nquad = len(steady) // 4

            def quad(q, carry):
                j0 = 2 + 4 * q
                for r in range(4):
                    j = j0 + r
                    b = (2 + r) % 4
                    wait_s(j - 2, r % 4)
                    issue_g(j + 2, r % 4)
                    wait_g(j, b)
                    issue_s(j, b)
                return carry

            lax.fori_loop(0, nquad, quad, 0)
            for j in range(2 + 4 * nquad, ch - 2):
                wait_s(j - 2, (j - 2) % 4)
                issue_g(j + 2, (j + 2) % 4)
                wait_g(j, j % 4)
                issue_s(j, j % 4)
            for j in range(ch - 2, ch):
                wait_g(j, j % 4)
                issue_s(j, j % 4)
            for j in range(ch - 4, ch):
                wait_s(j, j % 4)

        @pl.when(c == 0)
        def _():
            schedule(ch0)

        @pl.when(c == 1)
        def _():
            schedule(ch1)

        plsc.subcore_barrier()
        pltpu.sync_copy(
            acc.at[pl.ds(s * stripe, stripe)],
            out_hbm.at[c, pl.ds(s * stripe, stripe)],
        )

    return body


def _norm_factors(cnt_ref, hl_ref):
    cnt = cnt_ref[0, :] + cnt_ref[1, :]
    hl = hl_ref[0, :] + hl_ref[1, :]
    loopw = jnp.where(hl == 0.0, 1.0, 0.0)
    dis = lax.rsqrt(cnt + loopw)
    bn = cnt.shape[0]
    return jnp.reshape(dis, (bn, 1)), jnp.reshape(loopw, (bn, 1))


def _mm_scale_body(x_ref, w_ref, cnt_ref, hl_ref, yp_ref, st_ref):
    dis, loopw = _norm_factors(cnt_ref, hl_ref)
    y = jnp.dot(x_ref[:, :], w_ref[:, :], preferred_element_type=jnp.float32)
    yp_ref[:, :] = y * dis
    st_ref[:, :] = y * (loopw * dis * dis)


def _layer_body(agg_ref, st_ref, cnt_ref, hl_ref, b_ref, w_ref, yp_ref, st2_ref):
    dis, loopw = _norm_factors(cnt_ref, hl_ref)
    agg = agg_ref[0, :, :] + agg_ref[1, :, :]
    hcur = jnp.maximum(agg * dis + st_ref[:, :] + b_ref[:, :], 0.0)
    y2 = jnp.dot(hcur, w_ref[:, :], preferred_element_type=jnp.float32)
    yp_ref[:, :] = y2 * dis
    st2_ref[:, :] = y2 * (loopw * dis * dis)


def _head_body(agg_ref, st_ref, cnt_ref, hl_ref, b_ref, wl_ref, bl_ref, o_ref):
    dis, loopw = _norm_factors(cnt_ref, hl_ref)
    agg = agg_ref[0, :, :] + agg_ref[1, :, :]
    hcur = jnp.maximum(agg * dis + st_ref[:, :] + b_ref[:, :], 0.0)
    o = jnp.dot(hcur, wl_ref[:, :], preferred_element_type=jnp.float32) + bl_ref[:, :]
    m = jnp.max(o, axis=1, keepdims=True)
    lse = m + jnp.log(jnp.sum(jnp.exp(o - m), axis=1, keepdims=True))
    o_ref[:, :] = o - lse


def kernel(x, edge_index, W1, b1, W2, b2, Wl, bl):
    n, f_in = x.shape
    e = edge_index.shape[1]
    h = W1.shape[1]
    c_out = Wl.shape[1]
    assert n % NS == 0 and h % 16 == 0

    nch = -(-e // (NS * K))
    ch1 = max(8, int(round(nch / 3.5)))
    ch0 = nch - ch1
    e_pad = NS * nch * K
    acc_n = -(-(n + 1) // (NS * K)) * (NS * K)

    ei = edge_index.astype(jnp.int32)
    rows = jnp.concatenate([ei[0], jnp.zeros((e_pad - e,), jnp.int32)])
    cols = jnp.concatenate([ei[1], jnp.full((e_pad - e,), n, jnp.int32)])

    def _balanced(flat, fill):
        p0 = flat[: NS * ch0 * K].reshape(NS, ch0, K)
        p1 = flat[NS * ch0 * K :].reshape(NS, ch1, K)
        p1 = jnp.pad(p1, ((0, 0), (0, ch0 - ch1), (0, 0)), constant_values=fill)
        return jnp.stack([p0, p1], axis=1).reshape(NW, ch0, K)

    rows_l = _balanced(rows, 0)
    cols_l = _balanced(cols, n)

    mesh = _mesh()

    cnt0, hl0 = pl.kernel(
        _deg_body(ch0, ch1, acc_n, n),
        out_type=[
            jax.ShapeDtypeStruct((NC, acc_n), jnp.float32),
            jax.ShapeDtypeStruct((NC, acc_n), jnp.float32),
        ],
        mesh=mesh,
        scratch_types=[
            pltpu.VMEM((ch0, K), jnp.int32),
            pltpu.VMEM((ch0, K), jnp.int32),
            pltpu.VMEM((K,), jnp.float32),
            pltpu.VMEM((K,), jnp.float32),
            pltpu.VMEM_SHARED((acc_n,), jnp.float32),
            pltpu.VMEM_SHARED((acc_n,), jnp.float32),
        ],
    )(rows_l, cols_l)

    bn = 1280
    assert acc_n % bn == 0
    grid = (acc_n // bn,)
    y1p, st1 = pl.pallas_call(
        _mm_scale_body,
        grid=grid,
        in_specs=[
            pl.BlockSpec((bn, f_in), lambda i: (i, 0)),
            pl.BlockSpec((f_in, h), lambda i: (0, 0)),
            pl.BlockSpec((NC, bn), lambda i: (0, i)),
            pl.BlockSpec((NC, bn), lambda i: (0, i)),
        ],
        out_specs=[
            pl.BlockSpec((bn, h), lambda i: (i, 0)),
            pl.BlockSpec((bn, h), lambda i: (i, 0)),
        ],
        out_shape=[
            jax.ShapeDtypeStruct((acc_n, h), jnp.float32),
            jax.ShapeDtypeStruct((acc_n, h), jnp.float32),
        ],
    )(x, W1, cnt0, hl0)

    agg_call = pl.kernel(
        _agg_body(ch0, ch1, acc_n, n, h),
        out_type=jax.ShapeDtypeStruct((NC, acc_n, h), jnp.float32),
        mesh=mesh,
        compiler_params=pltpu.CompilerParams(use_tc_tiling_on_sc=False),
        scratch_types=[
            pltpu.VMEM((ch0, K), jnp.int32),
            pltpu.VMEM((ch0, K), jnp.int32),
            pltpu.VMEM((4, K, h), jnp.float32),
            pltpu.VMEM_SHARED((acc_n, h), jnp.float32),
        ]
        + [pltpu.SemaphoreType.DMA] * 8,
    )

    agg1 = agg_call(y1p, rows_l, cols_l)

    y2p, st2 = pl.pallas_call(
        _layer_body,
        grid=grid,
        in_specs=[
            pl.BlockSpec((NC, bn, h), lambda i: (0, i, 0)),
            pl.BlockSpec((bn, h), lambda i: (i, 0)),
            pl.BlockSpec((NC, bn), lambda i: (0, i)),
            pl.BlockSpec((NC, bn), lambda i: (0, i)),
            pl.BlockSpec((1, h), lambda i: (0, 0)),
            pl.BlockSpec((h, h), lambda i: (0, 0)),
        ],
        out_specs=[
            pl.BlockSpec((bn, h), lambda i: (i, 0)),
            pl.BlockSpec((bn, h), lambda i: (i, 0)),
        ],
        out_shape=[
            jax.ShapeDtypeStruct((acc_n, h), jnp.float32),
            jax.ShapeDtypeStruct((acc_n, h), jnp.float32),
        ],
    )(agg1, st1, cnt0, hl0, b1.reshape(1, h), W2)

    agg2 = agg_call(y2p, rows_l, cols_l)

    out = pl.pallas_call(
        _head_body,
        grid=grid,
        in_specs=[
            pl.BlockSpec((NC, bn, h), lambda i: (0, i, 0)),
            pl.BlockSpec((bn, h), lambda i: (i, 0)),
            pl.BlockSpec((NC, bn), lambda i: (0, i)),
            pl.BlockSpec((NC, bn), lambda i: (0, i)),
            pl.BlockSpec((1, h), lambda i: (0, 0)),
            pl.BlockSpec((h, c_out), lambda i: (0, 0)),
            pl.BlockSpec((1, c_out), lambda i: (0, 0)),
        ],
        out_specs=pl.BlockSpec((bn, c_out), lambda i: (i, 0)),
        out_shape=jax.ShapeDtypeStruct((n, c_out), jnp.float32),
    )(agg2, st2, cnt0, hl0, b2.reshape(1, h), Wl, bl.reshape(1, c_out))

    return out


# speedup vs baseline: 1.5921x; 1.1008x over previous
import jax
import jax.numpy as jnp
from jax import lax
from jax.experimental import pallas as pl
from jax.experimental.pallas import tpu as pltpu
from jax.experimental.pallas import tpu_sc as plsc

NC = 2
NS = 16
NW = NC * NS
K = 128


def _mesh():
    return plsc.VectorSubcoreMesh(
        core_axis_name="c", subcore_axis_name="s", num_cores=NC, num_subcores=NS
    )


def _deg_body(ch0, ch1, acc_n, n):
    stripe = acc_n // NS

    def body(rows_hbm, cols_hbm, cnt_hbm, hl_hbm, ridx, cidx, vones, vhl, acc_c, acc_h):
        c = lax.axis_index("c")
        s = lax.axis_index("s")
        wid = c * NS + s
        cha = -(-(ch0 + ch1 + 1) // 2 // 8) * 8
        chb = ch0 + ch1 - cha
        ch_c = jnp.where(c == 0, cha, chb)
        zeros16 = jnp.zeros((16,), jnp.float32)
        ones16 = jnp.ones((16,), jnp.float32)

        def zv(i, carry):
            vhl[pl.ds(i * 16, 16)] = zeros16
            return carry

        lax.fori_loop(0, K // 16, zv, 0)
        for t in range(stripe // K):
            pltpu.sync_copy(vhl, acc_c.at[pl.ds(s * stripe + t * K, K)])
            pltpu.sync_copy(vhl, acc_h.at[pl.ds(s * stripe + t * K, K)])

        def sv(i, carry):
            vones[pl.ds(i * 16, 16)] = ones16
            return carry

        lax.fori_loop(0, K // 16, sv, 0)
        plsc.subcore_barrier()

        pltpu.sync_copy(rows_hbm.at[wid], ridx)
        pltpu.sync_copy(cols_hbm.at[wid], cidx)

        @pl.when(c == 1)
        def _():
            pltpu.sync_copy(
                rows_hbm.at[s, pl.ds(cha, ch0 - cha)],
                ridx.at[pl.ds(ch1, ch0 - cha)],
            )
            pltpu.sync_copy(
                cols_hbm.at[s, pl.ds(cha, ch0 - cha)],
                cidx.at[pl.ds(ch1, ch0 - cha)],
            )

        def chunk(j, carry):
            def inner(i, icarry):
                r16 = ridx[j, pl.ds(i * 16, 16)]
                c16 = cidx[j, pl.ds(i * 16, 16)]
                vhl[pl.ds(i * 16, 16)] = jnp.where(r16 == c16, ones16, zeros16)
                return icarry

            lax.fori_loop(0, K // 16, inner, 0)
            pltpu.sync_copy(vones, acc_c.at[cidx.at[j]], add=True)
            pltpu.sync_copy(vhl, acc_h.at[cidx.at[j]], add=True)
            return carry

        lax.fori_loop(0, ch_c, chunk, 0)
        plsc.subcore_barrier()

        pltpu.sync_copy(
            acc_c.at[pl.ds(s * stripe, stripe)],
            cnt_hbm.at[c, pl.ds(s * stripe, stripe)],
        )
        pltpu.sync_copy(
            acc_h.at[pl.ds(s * stripe, stripe)],
            hl_hbm.at[c, pl.ds(s * stripe, stripe)],
        )

    return body


def _agg_body(ch0, ch1, acc_n, n, h):
    stripe = acc_n // NS

    assert ch0 >= 8 and ch1 >= 8

    def body(
        yp_hbm, rows_hbm, cols_hbm, out_hbm, ridx, cidx, gbuf, acc,
        g0, g1, g2, g3, s0, s1, s2, s3,
    ):
        c = lax.axis_index("c")
        s = lax.axis_index("s")
        wid = c * NS + s
        zeros16 = jnp.zeros((16,), jnp.float32)
        gsems = (g0, g1, g2, g3)
        ssems = (s0, s1, s2, s3)

        def zg(i, carry):
            gbuf[0, i // (h // 16), pl.ds((i % (h // 16)) * 16, 16)] = zeros16
            return carry

        lax.fori_loop(0, K * h // 16, zg, 0)

        for t in range(stripe // K):
            pltpu.sync_copy(gbuf.at[0], acc.at[pl.ds(s * stripe + t * K, K)])
        plsc.subcore_barrier()

        pltpu.sync_copy(rows_hbm.at[wid], ridx)
        pltpu.sync_copy(cols_hbm.at[wid], cidx)

        def issue_g(j, b):
            pltpu.async_copy(yp_hbm.at[ridx.at[j]], gbuf.at[b], gsems[b])

        def wait_g(j, b):
            pltpu.make_async_copy(yp_hbm.at[ridx.at[j]], gbuf.at[b], gsems[b]).wait()

        def issue_s(j, b):
            pltpu.async_copy(gbuf.at[b], acc.at[cidx.at[j]], ssems[b], add=True)

        def wait_s(j, b):
            pltpu.make_async_copy(gbuf.at[b], acc.at[cidx.at[j]], ssems[b]).wait()

        def schedule(ch):
            issue_g(0, 0)
            issue_g(1, 1)
            issue_g(2, 2)
            wait_g(0, 0)
            issue_s(0, 0)
            issue_g(3, 3)
            wait_g(1, 1)
            issue_s(1, 1)

            steady = range(2, ch - 2)
            nquad = len(steady) // 4

            def quad(q, carry):
                j0 = 2 + 4 * q
                for r in range(4):
                    j = j0 + r
                    b = (2 + r) % 4
                    wait_s(j - 2, r % 4)
                    issue_g(j + 2, r % 4)
                    wait_g(j, b)
                    issue_s(j, b)
                return carry

            lax.fori_loop(0, nquad, quad, 0)
            for j in range(2 + 4 * nquad, ch - 2):
                wait_s(j - 2, (j - 2) % 4)
                issue_g(j + 2, (j + 2) % 4)
                wait_g(j, j % 4)
                issue_s(j, j % 4)
            for j in range(ch - 2, ch):
                wait_g(j, j % 4)
                issue_s(j, j % 4)
            for j in range(ch - 4, ch):
                wait_s(j, j % 4)

        @pl.when(c == 0)
        def _():
            schedule(ch0)

        @pl.when(c == 1)
        def _():
            schedule(ch1)

        plsc.subcore_barrier()
        pltpu.sync_copy(
            acc.at[pl.ds(s * stripe, stripe)],
            out_hbm.at[c, pl.ds(s * stripe, stripe), pl.ds(0, h)],
        )

    return body


def _norm_factors(cnt_ref, hl_ref):
    cnt = cnt_ref[0, :] + cnt_ref[1, :]
    hl = hl_ref[0, :] + hl_ref[1, :]
    loopw = jnp.where(hl == 0.0, 1.0, 0.0)
    dis = lax.rsqrt(cnt + loopw)
    bn = cnt.shape[0]
    return jnp.reshape(dis, (bn, 1)), jnp.reshape(loopw, (bn, 1))


def _mm_scale_body(x_ref, w_ref, cnt_ref, hl_ref, yp_ref, st_ref):
    dis, loopw = _norm_factors(cnt_ref, hl_ref)
    y = jnp.dot(x_ref[:, :], w_ref[:, :], preferred_element_type=jnp.float32)
    yp_ref[:, :] = y * dis
    st_ref[:, :] = y * (loopw * dis * dis)


def _layer_body(agg_ref, st_ref, cnt_ref, hl_ref, b_ref, w_ref, yp_ref, st2_ref):
    dis, loopw = _norm_factors(cnt_ref, hl_ref)
    hh = st_ref.shape[1]
    agg = agg_ref[0, :, :hh] + agg_ref[1, :, :hh]
    hcur = jnp.maximum(agg * dis + st_ref[:, :] + b_ref[:, :], 0.0)
    y2 = jnp.dot(hcur, w_ref[:, :], preferred_element_type=jnp.float32)
    yp_ref[:, :] = y2 * dis
    st2_ref[:, :] = y2 * (loopw * dis * dis)


def _head_body(agg_ref, st_ref, cnt_ref, hl_ref, b_ref, wl_ref, bl_ref, o_ref):
    dis, loopw = _norm_factors(cnt_ref, hl_ref)
    hh = st_ref.shape[1]
    agg = agg_ref[0, :, :hh] + agg_ref[1, :, :hh]
    hcur = jnp.maximum(agg * dis + st_ref[:, :] + b_ref[:, :], 0.0)
    o = jnp.dot(hcur, wl_ref[:, :], preferred_element_type=jnp.float32) + bl_ref[:, :]
    m = jnp.max(o, axis=1, keepdims=True)
    lse = m + jnp.log(jnp.sum(jnp.exp(o - m), axis=1, keepdims=True))
    o_ref[:, :] = o - lse


def kernel(x, edge_index, W1, b1, W2, b2, Wl, bl):
    n, f_in = x.shape
    e = edge_index.shape[1]
    h = W1.shape[1]
    c_out = Wl.shape[1]
    assert n % NS == 0 and h % 16 == 0

    nch = -(-e // (NS * K))
    ch1 = max(8, int(round(nch / 3.5)))
    ch0 = nch - ch1
    e_pad = NS * nch * K
    acc_n = -(-(n + 1) // (NS * K)) * (NS * K)

    ei = edge_index.astype(jnp.int32)
    rows = jnp.concatenate([ei[0], jnp.zeros((e_pad - e,), jnp.int32)])
    cols = jnp.concatenate([ei[1], jnp.full((e_pad - e,), n, jnp.int32)])

    def _balanced(flat, fill):
        p0 = flat[: NS * ch0 * K].reshape(NS, ch0, K)
        p1 = flat[NS * ch0 * K :].reshape(NS, ch1, K)
        p1 = jnp.pad(p1, ((0, 0), (0, ch0 - ch1), (0, 0)), constant_values=fill)
        return jnp.concatenate([p0, p1], axis=0)

    rows_l = _balanced(rows, 0)
    cols_l = _balanced(cols, n)

    mesh = _mesh()

    cnt0, hl0 = pl.kernel(
        _deg_body(ch0, ch1, acc_n, n),
        out_type=[
            jax.ShapeDtypeStruct((NC, acc_n), jnp.float32),
            jax.ShapeDtypeStruct((NC, acc_n), jnp.float32),
        ],
        mesh=mesh,
        scratch_types=[
            pltpu.VMEM((ch0, K), jnp.int32),
            pltpu.VMEM((ch0, K), jnp.int32),
            pltpu.VMEM((K,), jnp.float32),
            pltpu.VMEM((K,), jnp.float32),
            pltpu.VMEM_SHARED((acc_n,), jnp.float32),
            pltpu.VMEM_SHARED((acc_n,), jnp.float32),
        ],
    )(rows_l, cols_l)

    bn = 1280
    assert acc_n % bn == 0
    grid = (acc_n // bn,)
    y1p, st1 = pl.pallas_call(
        _mm_scale_body,
        grid=grid,
        in_specs=[
            pl.BlockSpec((bn, f_in), lambda i: (i, 0)),
            pl.BlockSpec((f_in, h), lambda i: (0, 0)),
            pl.BlockSpec((NC, bn), lambda i: (0, i)),
            pl.BlockSpec((NC, bn), lambda i: (0, i)),
        ],
        out_specs=[
            pl.BlockSpec((bn, h), lambda i: (i, 0)),
            pl.BlockSpec((bn, h), lambda i: (i, 0)),
        ],
        out_shape=[
            jax.ShapeDtypeStruct((acc_n, h), jnp.float32),
            jax.ShapeDtypeStruct((acc_n, h), jnp.float32),
        ],
    )(x, W1, cnt0, hl0)

    agg_call = pl.kernel(
        _agg_body(ch0, ch1, acc_n, n, h),
        out_type=jax.ShapeDtypeStruct((NC, acc_n, 2 * h), jnp.float32),
        mesh=mesh,
        compiler_params=pltpu.CompilerParams(use_tc_tiling_on_sc=False),
        scratch_types=[
            pltpu.VMEM((ch0, K), jnp.int32),
            pltpu.VMEM((ch0, K), jnp.int32),
            pltpu.VMEM((4, K, h), jnp.float32),
            pltpu.VMEM_SHARED((acc_n, h), jnp.float32),
        ]
        + [pltpu.SemaphoreType.DMA] * 8,
    )

    agg1 = agg_call(y1p, rows_l, cols_l)

    y2p, st2 = pl.pallas_call(
        _layer_body,
        grid=grid,
        in_specs=[
            pl.BlockSpec((NC, bn, 2 * h), lambda i: (0, i, 0)),
            pl.BlockSpec((bn, h), lambda i: (i, 0)),
            pl.BlockSpec((NC, bn), lambda i: (0, i)),
            pl.BlockSpec((NC, bn), lambda i: (0, i)),
            pl.BlockSpec((1, h), lambda i: (0, 0)),
            pl.BlockSpec((h, h), lambda i: (0, 0)),
        ],
        out_specs=[
            pl.BlockSpec((bn, h), lambda i: (i, 0)),
            pl.BlockSpec((bn, h), lambda i: (i, 0)),
        ],
        out_shape=[
            jax.ShapeDtypeStruct((acc_n, h), jnp.float32),
            jax.ShapeDtypeStruct((acc_n, h), jnp.float32),
        ],
    )(agg1, st1, cnt0, hl0, b1.reshape(1, h), W2)

    agg2 = agg_call(y2p, rows_l, cols_l)

    out = pl.pallas_call(
        _head_body,
        grid=grid,
        in_specs=[
            pl.BlockSpec((NC, bn, 2 * h), lambda i: (0, i, 0)),
            pl.BlockSpec((bn, h), lambda i: (i, 0)),
            pl.BlockSpec((NC, bn), lambda i: (0, i)),
            pl.BlockSpec((NC, bn), lambda i: (0, i)),
            pl.BlockSpec((1, h), lambda i: (0, 0)),
            pl.BlockSpec((h, c_out), lambda i: (0, 0)),
            pl.BlockSpec((1, c_out), lambda i: (0, 0)),
        ],
        out_specs=pl.BlockSpec((bn, c_out), lambda i: (i, 0)),
        out_shape=jax.ShapeDtypeStruct((n, c_out), jnp.float32),
    )(agg2, st2, cnt0, hl0, b2.reshape(1, h), Wl, bl.reshape(1, c_out))

    return out


# speedup vs baseline: 1.6101x; 1.0113x over previous
import jax
import jax.numpy as jnp
from jax import lax
from jax.experimental import pallas as pl
from jax.experimental.pallas import tpu as pltpu
from jax.experimental.pallas import tpu_sc as plsc

NC = 2
NS = 16
NW = NC * NS
K = 128


def _mesh():
    return plsc.VectorSubcoreMesh(
        core_axis_name="c", subcore_axis_name="s", num_cores=NC, num_subcores=NS
    )


def _deg_body(ch0, ch1, acc_n, n):
    stripe = acc_n // NS

    def body(rows_hbm, cols_hbm, cnt_hbm, hl_hbm, ridx, cidx, vones, vhl, acc_c, acc_h):
        c = lax.axis_index("c")
        s = lax.axis_index("s")
        wid = c * NS + s
        cha = -(-(ch0 + ch1 + 1) // 2 // 8) * 8
        chb = ch0 + ch1 - cha
        ch_c = jnp.where(c == 0, cha, chb)
        zeros16 = jnp.zeros((16,), jnp.float32)
        ones16 = jnp.ones((16,), jnp.float32)

        def zv(i, carry):
            vhl[pl.ds(i * 16, 16)] = zeros16
            return carry

        lax.fori_loop(0, K // 16, zv, 0)
        for t in range(stripe // K):
            pltpu.sync_copy(vhl, acc_c.at[pl.ds(s * stripe + t * K, K)])
            pltpu.sync_copy(vhl, acc_h.at[pl.ds(s * stripe + t * K, K)])

        def sv(i, carry):
            vones[pl.ds(i * 16, 16)] = ones16
            return carry

        lax.fori_loop(0, K // 16, sv, 0)
        plsc.subcore_barrier()

        pltpu.sync_copy(rows_hbm.at[wid], ridx)
        pltpu.sync_copy(cols_hbm.at[wid], cidx)

        @pl.when(c == 1)
        def _():
            pltpu.sync_copy(
                rows_hbm.at[s, pl.ds(cha, ch0 - cha)],
                ridx.at[pl.ds(ch1, ch0 - cha)],
            )
            pltpu.sync_copy(
                cols_hbm.at[s, pl.ds(cha, ch0 - cha)],
                cidx.at[pl.ds(ch1, ch0 - cha)],
            )

        def chunk(j, carry):
            def inner(i, icarry):
                r16 = ridx[j, pl.ds(i * 16, 16)]
                c16 = cidx[j, pl.ds(i * 16, 16)]
                vhl[pl.ds(i * 16, 16)] = jnp.where(r16 == c16, ones16, zeros16)
                return icarry

            lax.fori_loop(0, K // 16, inner, 0)
            pltpu.sync_copy(vones, acc_c.at[cidx.at[j]], add=True)
            pltpu.sync_copy(vhl, acc_h.at[cidx.at[j]], add=True)
            return carry

        lax.fori_loop(0, ch_c, chunk, 0)
        plsc.subcore_barrier()

        pltpu.sync_copy(
            acc_c.at[pl.ds(s * stripe, stripe)],
            cnt_hbm.at[c, pl.ds(s * stripe, stripe)],
        )
        pltpu.sync_copy(
            acc_h.at[pl.ds(s * stripe, stripe)],
            hl_hbm.at[c, pl.ds(s * stripe, stripe)],
        )

    return body


def _agg_body(ch0, ch1, acc_n, n, h):
    stripe = acc_n // NS

    assert ch0 >= 8 and ch1 >= 8

    def body(
        yp_hbm, rows_hbm, cols_hbm, out_hbm, ridx, cidxa, cidxb, gbuf, acc,
        g0, g1, g2, g3, s0, s1, s2, s3, t0, t1, t2, t3,
    ):
        c = lax.axis_index("c")
        s = lax.axis_index("s")
        wid = c * NS + s
        zeros16 = jnp.zeros((16,), jnp.float32)
        gsems = (g0, g1, g2, g3)
        ssems = (s0, s1, s2, s3)
        tsems = (t0, t1, t2, t3)

        def zg(i, carry):
            gbuf[0, i // (h // 16), pl.ds((i % (h // 16)) * 16, 16)] = zeros16
            return carry

        lax.fori_loop(0, K * h // 16, zg, 0)

        for t in range(stripe // K):
            pltpu.sync_copy(gbuf.at[0], acc.at[pl.ds(s * stripe + t * K, K)])
        plsc.subcore_barrier()

        pltpu.sync_copy(rows_hbm.at[wid], ridx)
        pltpu.sync_copy(cols_hbm.at[wid, :, pl.ds(0, K // 2)], cidxa)
        pltpu.sync_copy(cols_hbm.at[wid, :, pl.ds(K // 2, K // 2)], cidxb)

        def issue_g(j, b):
            pltpu.async_copy(yp_hbm.at[ridx.at[j]], gbuf.at[b], gsems[b])

        def wait_g(j, b):
            pltpu.make_async_copy(yp_hbm.at[ridx.at[j]], gbuf.at[b], gsems[b]).wait()

        def issue_s(j, b):
            pltpu.async_copy(
                gbuf.at[b, pl.ds(0, K // 2)], acc.at[cidxa.at[j]], ssems[b],
                add=True,
            )
            pltpu.async_copy(
                gbuf.at[b, pl.ds(K // 2, K // 2)], acc.at[cidxb.at[j]], tsems[b],
                add=True,
            )

        def wait_s(j, b):
            pltpu.make_async_copy(
                gbuf.at[b, pl.ds(0, K // 2)], acc.at[cidxa.at[j]], ssems[b]
            ).wait()
            pltpu.make_async_copy(
                gbuf.at[b, pl.ds(K // 2, K // 2)], acc.at[cidxb.at[j]], tsems[b]
            ).wait()

        def schedule(ch):
            issue_g(0, 0)
            issue_g(1, 1)
            issue_g(2, 2)
            wait_g(0, 0)
            issue_s(0, 0)
            issue_g(3, 3)
            wait_g(1, 1)
            issue_s(1, 1)

            steady = range(2, ch - 2)
            nquad = len(steady) // 4

            def quad(q, carry):
                j0 = 2 + 4 * q
                for r in range(4):
                    j = j0 + r
                    b = (2 + r) % 4
                    wait_s(j - 2, r % 4)
                    issue_g(j + 2, r % 4)
                    wait_g(j, b)
                    issue_s(j, b)
                return carry

            lax.fori_loop(0, nquad, quad, 0)
            for j in range(2 + 4 * nquad, ch - 2):
                wait_s(j - 2, (j - 2) % 4)
                issue_g(j + 2, (j + 2) % 4)
                wait_g(j, j % 4)
                issue_s(j, j % 4)
            for j in range(ch - 2, ch):
                wait_g(j, j % 4)
                issue_s(j, j % 4)
            for j in range(ch - 4, ch):
                wait_s(j, j % 4)

        @pl.when(c == 0)
        def _():
            schedule(ch0)

        @pl.when(c == 1)
        def _():
            schedule(ch1)

        plsc.subcore_barrier()
        pltpu.sync_copy(
            acc.at[pl.ds(s * stripe, stripe)],
            out_hbm.at[c, pl.ds(s * stripe, stripe), pl.ds(0, h)],
        )

    return body


def _norm_factors(cnt_ref, hl_ref):
    cnt = cnt_ref[0, :] + cnt_ref[1, :]
    hl = hl_ref[0, :] + hl_ref[1, :]
    loopw = jnp.where(hl == 0.0, 1.0, 0.0)
    dis = lax.rsqrt(cnt + loopw)
    bn = cnt.shape[0]
    return jnp.reshape(dis, (bn, 1)), jnp.reshape(loopw, (bn, 1))


def _mm_scale_body(x_ref, w_ref, cnt_ref, hl_ref, yp_ref, st_ref):
    dis, loopw = _norm_factors(cnt_ref, hl_ref)
    y = jnp.dot(x_ref[:, :], w_ref[:, :], preferred_element_type=jnp.float32)
    yp_ref[:, :] = y * dis
    st_ref[:, :] = y * (loopw * dis * dis)


def _layer_body(agg_ref, st_ref, cnt_ref, hl_ref, b_ref, w_ref, yp_ref, st2_ref):
    dis, loopw = _norm_factors(cnt_ref, hl_ref)
    hh = st_ref.shape[1]
    agg = agg_ref[0, :, :hh] + agg_ref[1, :, :hh]
    hcur = jnp.maximum(agg * dis + st_ref[:, :] + b_ref[:, :], 0.0)
    y2 = jnp.dot(hcur, w_ref[:, :], preferred_element_type=jnp.float32)
    yp_ref[:, :] = y2 * dis
    st2_ref[:, :] = y2 * (loopw * dis * dis)


def _head_body(agg_ref, st_ref, cnt_ref, hl_ref, b_ref, wl_ref, bl_ref, o_ref):
    dis, loopw = _norm_factors(cnt_ref, hl_ref)
    hh = st_ref.shape[1]
    agg = agg_ref[0, :, :hh] + agg_ref[1, :, :hh]
    hcur = jnp.maximum(agg * dis + st_ref[:, :] + b_ref[:, :], 0.0)
    o = jnp.dot(hcur, wl_ref[:, :], preferred_element_type=jnp.float32) + bl_ref[:, :]
    m = jnp.max(o, axis=1, keepdims=True)
    lse = m + jnp.log(jnp.sum(jnp.exp(o - m), axis=1, keepdims=True))
    o_ref[:, :] = o - lse


def kernel(x, edge_index, W1, b1, W2, b2, Wl, bl):
    n, f_in = x.shape
    e = edge_index.shape[1]
    h = W1.shape[1]
    c_out = Wl.shape[1]
    assert n % NS == 0 and h % 16 == 0

    nch = -(-e // (NS * K))
    ch1 = max(8, int(round(nch / 3.5)))
    ch0 = nch - ch1
    e_pad = NS * nch * K
    acc_n = -(-(n + 1) // (NS * K)) * (NS * K)

    ei = edge_index.astype(jnp.int32)
    rows = jnp.concatenate([ei[0], jnp.zeros((e_pad - e,), jnp.int32)])
    cols = jnp.concatenate([ei[1], jnp.full((e_pad - e,), n, jnp.int32)])

    def _balanced(flat, fill):
        p0 = flat[: NS * ch0 * K].reshape(NS, ch0, K)
        p1 = flat[NS * ch0 * K :].reshape(NS, ch1, K)
        p1 = jnp.pad(p1, ((0, 0), (0, ch0 - ch1), (0, 0)), constant_values=fill)
        return jnp.concatenate([p0, p1], axis=0)

    rows_l = _balanced(rows, 0)
    cols_l = _balanced(cols, n)

    mesh = _mesh()

    cnt0, hl0 = pl.kernel(
        _deg_body(ch0, ch1, acc_n, n),
        out_type=[
            jax.ShapeDtypeStruct((NC, acc_n), jnp.float32),
            jax.ShapeDtypeStruct((NC, acc_n), jnp.float32),
        ],
        mesh=mesh,
        scratch_types=[
            pltpu.VMEM((ch0, K), jnp.int32),
            pltpu.VMEM((ch0, K), jnp.int32),
            pltpu.VMEM((K,), jnp.float32),
            pltpu.VMEM((K,), jnp.float32),
            pltpu.VMEM_SHARED((acc_n,), jnp.float32),
            pltpu.VMEM_SHARED((acc_n,), jnp.float32),
        ],
    )(rows_l, cols_l)

    bn = 1280
    assert acc_n % bn == 0
    grid = (acc_n // bn,)
    y1p, st1 = pl.pallas_call(
        _mm_scale_body,
        grid=grid,
        in_specs=[
            pl.BlockSpec((bn, f_in), lambda i: (i, 0)),
            pl.BlockSpec((f_in, h), lambda i: (0, 0)),
            pl.BlockSpec((NC, bn), lambda i: (0, i)),
            pl.BlockSpec((NC, bn), lambda i: (0, i)),
        ],
        out_specs=[
            pl.BlockSpec((bn, h), lambda i: (i, 0)),
            pl.BlockSpec((bn, h), lambda i: (i, 0)),
        ],
        out_shape=[
            jax.ShapeDtypeStruct((acc_n, h), jnp.float32),
            jax.ShapeDtypeStruct((acc_n, h), jnp.float32),
        ],
    )(x, W1, cnt0, hl0)

    agg_call = pl.kernel(
        _agg_body(ch0, ch1, acc_n, n, h),
        out_type=jax.ShapeDtypeStruct((NC, acc_n, 2 * h), jnp.float32),
        mesh=mesh,
        compiler_params=pltpu.CompilerParams(use_tc_tiling_on_sc=False),
        scratch_types=[
            pltpu.VMEM((ch0, K), jnp.int32),
            pltpu.VMEM((ch0, K // 2), jnp.int32),
            pltpu.VMEM((ch0, K // 2), jnp.int32),
            pltpu.VMEM((4, K, h), jnp.float32),
            pltpu.VMEM_SHARED((acc_n, h), jnp.float32),
        ]
        + [pltpu.SemaphoreType.DMA] * 12,
    )

    agg1 = agg_call(y1p, rows_l, cols_l)

    y2p, st2 = pl.pallas_call(
        _layer_body,
        grid=grid,
        in_specs=[
            pl.BlockSpec((NC, bn, 2 * h), lambda i: (0, i, 0)),
            pl.BlockSpec((bn, h), lambda i: (i, 0)),
            pl.BlockSpec((NC, bn), lambda i: (0, i)),
            pl.BlockSpec((NC, bn), lambda i: (0, i)),
            pl.BlockSpec((1, h), lambda i: (0, 0)),
            pl.BlockSpec((h, h), lambda i: (0, 0)),
        ],
        out_specs=[
            pl.BlockSpec((bn, h), lambda i: (i, 0)),
            pl.BlockSpec((bn, h), lambda i: (i, 0)),
        ],
        out_shape=[
            jax.ShapeDtypeStruct((acc_n, h), jnp.float32),
            jax.ShapeDtypeStruct((acc_n, h), jnp.float32),
        ],
    )(agg1, st1, cnt0, hl0, b1.reshape(1, h), W2)

    agg2 = agg_call(y2p, rows_l, cols_l)

    out = pl.pallas_call(
        _head_body,
        grid=grid,
        in_specs=[
            pl.BlockSpec((NC, bn, 2 * h), lambda i: (0, i, 0)),
            pl.BlockSpec((bn, h), lambda i: (i, 0)),
            pl.BlockSpec((NC, bn), lambda i: (0, i)),
            pl.BlockSpec((NC, bn), lambda i: (0, i)),
            pl.BlockSpec((1, h), lambda i: (0, 0)),
            pl.BlockSpec((h, c_out), lambda i: (0, 0)),
            pl.BlockSpec((1, c_out), lambda i: (0, 0)),
        ],
        out_specs=pl.BlockSpec((bn, c_out), lambda i: (i, 0)),
        out_shape=jax.ShapeDtypeStruct((n, c_out), jnp.float32),
    )(agg2, st2, cnt0, hl0, b2.reshape(1, h), Wl, bl.reshape(1, c_out))

    return out
